# baseline (speedup 1.0000x reference)
import functools

import jax
import jax.numpy as jnp
from jax import lax
from jax.experimental import pallas as pl
from jax.experimental.pallas import tpu as pltpu
from jax.experimental.pallas import tpu_sc as plsc

NC, NS = 2, 16
NW = NC * NS
SAMP = 16
H = 50
HP = 56
HP2 = 2 * HP
LP = 128


def _emb_body(x_hbm, table_hbm, out_hbm, idx0, idx1, rows0, rows1,
              sem_g0, sem_g1, sem_o0, sem_o1):
    wid = lax.axis_index("s") * NC + lax.axis_index("c")
    n_samples = x_hbm.shape[0]
    per_w = n_samples // NW
    n_chunks = per_w // SAMP
    base = wid * per_w

    idx_b = (idx0, idx1)
    rows_b = (rows0, rows1)
    sem_gb = (sem_g0, sem_g1)
    sem_ob = (sem_o0, sem_o1)

    def half(g, b):
        idx_v, rows_v = idx_b[b], rows_b[b]
        sem_g, sem_o = sem_gb[b], sem_ob[b]
        b0 = base + g * SAMP

        @pl.when(g >= 2)
        def _():
            for h in range(2):
                pltpu.make_async_copy(
                    rows_v.at[:, pl.ds(h * HP, HP), :],
                    out_hbm.at[pl.ds(0, 8), pl.ds(0, HP), pl.ds(0, 32)],
                    sem_o,
                ).wait()

        pltpu.sync_copy(
            x_hbm.at[pl.ds(b0, 8), pl.ds(0, HP)],
            idx_v.at[:, pl.ds(0, HP)],
        )
        pltpu.sync_copy(
            x_hbm.at[pl.ds(b0 + 8, 8), pl.ds(0, HP)],
            idx_v.at[:, pl.ds(HP, HP)],
        )
        gathers = [
            pltpu.async_copy(
                table_hbm.at[idx_v.at[j]],
                rows_v.at[j],
                sem_g,
            )
            for j in range(8)
        ]
        for cp in gathers:
            cp.wait()
        for h in range(2):
            pltpu.async_copy(
                rows_v.at[:, pl.ds(h * HP, HP), :],
                out_hbm.at[pl.ds(b0 + 8 * h, 8), pl.ds(0, HP), pl.ds(0, 32)],
                sem_o,
            )

    def pair(p, carry):
        half(2 * p, 0)
        half(2 * p + 1, 1)
        return carry

    lax.fori_loop(0, n_chunks // 2, pair, None)

    for b in range(2):
        for h in range(2):
            pltpu.make_async_copy(
                rows_b[b].at[:, pl.ds(h * HP, HP), :],
                out_hbm.at[pl.ds(0, 8), pl.ds(0, HP), pl.ds(0, 32)],
                sem_ob[b],
            ).wait()


def kernel(x, table):
    B, Hx = x.shape
    V, D = table.shape

    xp = jnp.pad(x, ((0, 0), (0, LP - Hx)), mode="edge")

    grid_kernel = pl.kernel(
        _emb_body,
        out_type=jax.ShapeDtypeStruct((B, HP, LP), jnp.float32),
        mesh=plsc.VectorSubcoreMesh(
            core_axis_name="c", subcore_axis_name="s"
        ),
        scratch_types=[
            pltpu.VMEM((8, HP2), jnp.int32),
            pltpu.VMEM((8, HP2), jnp.int32),
            pltpu.VMEM((8, HP2, D), jnp.float32),
            pltpu.VMEM((8, HP2, D), jnp.float32),
            pltpu.SemaphoreType.DMA,
            pltpu.SemaphoreType.DMA,
            pltpu.SemaphoreType.DMA,
            pltpu.SemaphoreType.DMA,
        ],
        compiler_params=pltpu.CompilerParams(
            use_tc_tiling_on_sc=False, skip_device_barrier=True
        ),
    )
    out_pad = grid_kernel(xp, table)
    return out_pad[:, :Hx, :D]


# speedup vs baseline: 1.0352x; 1.0352x over previous
import functools

import jax
import jax.numpy as jnp
from jax import lax
from jax.experimental import pallas as pl
from jax.experimental.pallas import tpu as pltpu
from jax.experimental.pallas import tpu_sc as plsc

NC, NS = 2, 16
NW = NC * NS
SAMP = 16
H = 50
HP = 56
HP2 = 2 * HP
LP = 128


def _emb_body(x_hbm, table_hbm, out_hbm, idx0, idx1, rows0, rows1,
              sem_g0, sem_g1, sem_o0, sem_o1):
    wid = lax.axis_index("s") * NC + lax.axis_index("c")
    n_samples = x_hbm.shape[0]
    per_w = n_samples // NW
    n_chunks = per_w // SAMP
    base = wid * per_w

    idx_b = (idx0, idx1)
    rows_b = (rows0, rows1)
    sem_gb = (sem_g0, sem_g1)
    sem_ob = (sem_o0, sem_o1)

    def stage(g, b):
        b0 = base + g * SAMP
        pltpu.sync_copy(
            x_hbm.at[pl.ds(b0, 8), pl.ds(0, HP)],
            idx_b[b].at[:, pl.ds(0, HP)],
        )
        pltpu.sync_copy(
            x_hbm.at[pl.ds(b0 + 8, 8), pl.ds(0, HP)],
            idx_b[b].at[:, pl.ds(HP, HP)],
        )

    def half(g, b):
        idx_v, rows_v = idx_b[b], rows_b[b]
        sem_g, sem_o = sem_gb[b], sem_ob[b]
        b0 = base + g * SAMP

        @pl.when(g >= 2)
        def _():
            for h in range(2):
                pltpu.make_async_copy(
                    rows_v.at[:, pl.ds(h * HP, HP), :],
                    out_hbm.at[pl.ds(0, 8), pl.ds(0, HP), pl.ds(0, 32)],
                    sem_o,
                ).wait()

        gathers = [
            pltpu.async_copy(
                table_hbm.at[idx_v.at[j]],
                rows_v.at[j],
                sem_g,
            )
            for j in range(8)
        ]

        @pl.when(g + 1 < n_chunks)
        def _():
            stage(g + 1, 1 - b)

        for cp in gathers:
            cp.wait()
        for h in range(2):
            pltpu.async_copy(
                rows_v.at[:, pl.ds(h * HP, HP), :],
                out_hbm.at[pl.ds(b0 + 8 * h, 8), pl.ds(0, HP), pl.ds(0, 32)],
                sem_o,
            )

    def pair(p, carry):
        half(2 * p, 0)
        half(2 * p + 1, 1)
        return carry

    stage(0, 0)
    lax.fori_loop(0, n_chunks // 2, pair, None)

    for b in range(2):
        for h in range(2):
            pltpu.make_async_copy(
                rows_b[b].at[:, pl.ds(h * HP, HP), :],
                out_hbm.at[pl.ds(0, 8), pl.ds(0, HP), pl.ds(0, 32)],
                sem_ob[b],
            ).wait()


def kernel(x, table):
    B, Hx = x.shape
    V, D = table.shape

    xp = jnp.pad(x, ((0, 0), (0, LP - Hx)), mode="edge")

    grid_kernel = pl.kernel(
        _emb_body,
        out_type=jax.ShapeDtypeStruct((B, HP, LP), jnp.float32),
        mesh=plsc.VectorSubcoreMesh(
            core_axis_name="c", subcore_axis_name="s"
        ),
        scratch_types=[
            pltpu.VMEM((8, HP2), jnp.int32),
            pltpu.VMEM((8, HP2), jnp.int32),
            pltpu.VMEM((8, HP2, D), jnp.float32),
            pltpu.VMEM((8, HP2, D), jnp.float32),
            pltpu.SemaphoreType.DMA,
            pltpu.SemaphoreType.DMA,
            pltpu.SemaphoreType.DMA,
            pltpu.SemaphoreType.DMA,
        ],
        compiler_params=pltpu.CompilerParams(
            use_tc_tiling_on_sc=False, skip_device_barrier=True
        ),
    )
    out_pad = grid_kernel(xp, table)
    return out_pad[:, :Hx, :D]
